# baseline (speedup 1.0000x reference)
import functools

import jax
import jax.numpy as jnp
from jax import lax
from jax.experimental import pallas as pl
from jax.experimental.pallas import tpu as pltpu
from jax.experimental.pallas import tpu_sc as plsc

M_SEG = 16
A_MIN = 0.5
A_MAX = 5.0

NC = 2
NS = 16
LANES = 16
NW = NC * NS

N_ROWS = 32768
N_COLS = 512
ROWS_PER_W = N_ROWS // NW
CHUNK_ROWS = 32
N_CHUNKS = ROWS_PER_W // CHUNK_ROWS
N_PAIRS = N_CHUNKS // 2
VREGS_PER_CHUNK = CHUNK_ROWS * N_COLS // LANES
VREGS_PER_ROW = N_COLS // LANES
UNROLL = 8


def _sc_body(s_hbm, u_hbm, out_hbm, in0, in1, ot0, ot1, u_v,
             isem0, isem1, osem0, osem1):
    wid = lax.axis_index("s") * NC + lax.axis_index("c")
    base = wid * ROWS_PER_W
    in_bufs = (in0, in1)
    out_bufs = (ot0, ot1)
    in_sems = (isem0, isem1)
    out_sems = (osem0, osem1)

    pltpu.sync_copy(u_hbm, u_v)
    u = u_v[...]
    sig = 1.0 / (1.0 + jnp.exp(-u))
    a = A_MIN + (A_MAX - A_MIN) * sig
    delta = 1.0 / M_SEG
    jf = lax.iota(jnp.int32, LANES).astype(jnp.float32)
    csum = jnp.zeros((LANES,), jnp.float32)
    for i in range(M_SEG):
        csum = csum + jnp.where(jf >= float(i), a[i], 0.0)
    inv_y = 1.0 / (jnp.full((LANES,), csum[M_SEG - 1], jnp.float32) * delta)
    c_vec = a * inv_y
    d_vec = (csum - a * (jf + 1.0)) * delta * inv_y

    gather_dnums = lax.GatherDimensionNumbers(
        offset_dims=(), collapsed_slice_dims=(0,), start_index_map=(0,))

    def table_lookup(tab, j):
        return lax.gather(tab, j[:, None], gather_dnums, slice_sizes=(1,),
                          mode=lax.GatherScatterMode.PROMISE_IN_BOUNDS)

    def src_slice(g):
        return s_hbm.at[pl.ds(base + g * CHUNK_ROWS, CHUNK_ROWS), :]

    def dst_slice(g):
        return out_hbm.at[pl.ds(base + g * CHUNK_ROWS, CHUNK_ROWS), :]

    for b in range(2):
        pltpu.async_copy(src_slice(b), in_bufs[b], in_sems[b])

    def round_body(gg, carry):
        for b in range(2):
            g = gg * 2 + b
            in_b, out_b = in_bufs[b], out_bufs[b]
            pltpu.make_async_copy(src_slice(g), in_b, in_sems[b]).wait()

            @pl.when(gg > 0)
            def _():
                pltpu.make_async_copy(out_b, dst_slice(g - 2),
                                      out_sems[b]).wait()

            @plsc.parallel_loop(0, CHUNK_ROWS, 1)
            def _(r):
                for cb in range(VREGS_PER_ROW):
                    col = cb * LANES
                    s = in_b[r, pl.ds(col, LANES)]
                    j = (s * float(M_SEG)).astype(jnp.int32)
                    c = table_lookup(c_vec, j)
                    d = table_lookup(d_vec, j)
                    out_b[r, pl.ds(col, LANES)] = c * s + d

            @pl.when(gg < N_PAIRS - 1)
            def _():
                pltpu.async_copy(src_slice(g + 2), in_b, in_sems[b])

            pltpu.async_copy(out_b, dst_slice(g), out_sems[b])
        return carry

    lax.fori_loop(0, N_PAIRS, round_body, 0)

    for b in range(2):
        g = (N_PAIRS - 1) * 2 + b
        pltpu.make_async_copy(out_bufs[b], dst_slice(g), out_sems[b]).wait()


@functools.partial(jax.jit, static_argnames=())
def kernel(S, u_i):
    mesh = plsc.VectorSubcoreMesh(
        core_axis_name="c", subcore_axis_name="s",
        num_cores=NC, num_subcores=NS,
    )
    run = functools.partial(
        pl.kernel,
        out_type=jax.ShapeDtypeStruct((N_ROWS, N_COLS), jnp.float32),
        mesh=mesh,
        compiler_params=pltpu.CompilerParams(needs_layout_passes=False),
        scratch_types=[
            pltpu.VMEM((CHUNK_ROWS, N_COLS), jnp.float32),
            pltpu.VMEM((CHUNK_ROWS, N_COLS), jnp.float32),
            pltpu.VMEM((CHUNK_ROWS, N_COLS), jnp.float32),
            pltpu.VMEM((CHUNK_ROWS, N_COLS), jnp.float32),
            pltpu.VMEM((LANES,), jnp.float32),
            pltpu.SemaphoreType.DMA,
            pltpu.SemaphoreType.DMA,
            pltpu.SemaphoreType.DMA,
            pltpu.SemaphoreType.DMA,
        ],
    )(_sc_body)
    out = run(S.reshape(N_ROWS, N_COLS), u_i)
    return out.reshape(S.shape)


# speedup vs baseline: 2.1148x; 2.1148x over previous
import functools

import jax
import jax.numpy as jnp
from jax import lax
from jax.experimental import pallas as pl
from jax.experimental.pallas import tpu as pltpu
from jax.experimental.pallas import tpu_sc as plsc

M_SEG = 16
A_MIN = 0.5
A_MAX = 5.0

NC = 2
NS = 16
LANES = 16
NW = NC * NS

N_ROWS = 32768
N_COLS = 512
ROWS_PER_W = N_ROWS // NW
CHUNK_ROWS = 32
N_CHUNKS = ROWS_PER_W // CHUNK_ROWS
N_PAIRS = N_CHUNKS // 2
VREGS_PER_CHUNK = CHUNK_ROWS * N_COLS // LANES
VREGS_PER_ROW = N_COLS // LANES
UNROLL = 8


def _sc_body(s_hbm, u_hbm, out_hbm, in0, in1, ot0, ot1, u_v, dtab,
             isem0, isem1, osem0, osem1):
    wid = lax.axis_index("s") * NC + lax.axis_index("c")
    base = wid * ROWS_PER_W
    in_bufs = (in0, in1)
    out_bufs = (ot0, ot1)
    in_sems = (isem0, isem1)
    out_sems = (osem0, osem1)

    pltpu.sync_copy(u_hbm, u_v)
    u = u_v[...]
    sig = 1.0 / (1.0 + jnp.exp(-u))
    a = A_MIN + (A_MAX - A_MIN) * sig
    delta = 1.0 / M_SEG
    jf = lax.iota(jnp.int32, LANES).astype(jnp.float32)
    csum = jnp.zeros((LANES,), jnp.float32)
    for i in range(M_SEG):
        csum = csum + jnp.where(jf >= float(i), a[i], 0.0)
    inv_y = 1.0 / (jnp.full((LANES,), csum[M_SEG - 1], jnp.float32) * delta)
    c_vec = a * inv_y
    dtab[...] = (csum - a * (jf + 1.0)) * delta * inv_y

    gather_dnums = lax.GatherDimensionNumbers(
        offset_dims=(), collapsed_slice_dims=(0,), start_index_map=(0,))

    def table_lookup(tab, j):
        return lax.gather(tab, j[:, None], gather_dnums, slice_sizes=(1,),
                          mode=lax.GatherScatterMode.PROMISE_IN_BOUNDS)

    def src_slice(g):
        return s_hbm.at[pl.ds(base + g * CHUNK_ROWS, CHUNK_ROWS), :]

    def dst_slice(g):
        return out_hbm.at[pl.ds(base + g * CHUNK_ROWS, CHUNK_ROWS), :]

    for b in range(2):
        pltpu.async_copy(src_slice(b), in_bufs[b], in_sems[b])

    def round_body(gg, carry):
        for b in range(2):
            g = gg * 2 + b
            in_b, out_b = in_bufs[b], out_bufs[b]
            pltpu.make_async_copy(src_slice(g), in_b, in_sems[b]).wait()

            @pl.when(gg > 0)
            def _():
                pltpu.make_async_copy(out_b, dst_slice(g - 2),
                                      out_sems[b]).wait()

            @plsc.parallel_loop(0, VREGS_PER_CHUNK, 1, unroll=UNROLL)
            def _(i):
                r = i // VREGS_PER_ROW
                col = (i % VREGS_PER_ROW) * LANES
                s = in_b[r, pl.ds(col, LANES)]
                j = (s * float(M_SEG)).astype(jnp.int32)
                c = table_lookup(c_vec, j)
                d = plsc.load_gather(dtab, [j])
                out_b[r, pl.ds(col, LANES)] = c * s + d

            @pl.when(gg < N_PAIRS - 1)
            def _():
                pltpu.async_copy(src_slice(g + 2), in_b, in_sems[b])

            pltpu.async_copy(out_b, dst_slice(g), out_sems[b])
        return carry

    lax.fori_loop(0, N_PAIRS, round_body, 0)

    for b in range(2):
        g = (N_PAIRS - 1) * 2 + b
        pltpu.make_async_copy(out_bufs[b], dst_slice(g), out_sems[b]).wait()


@functools.partial(jax.jit, static_argnames=())
def kernel(S, u_i):
    mesh = plsc.VectorSubcoreMesh(
        core_axis_name="c", subcore_axis_name="s",
        num_cores=NC, num_subcores=NS,
    )
    run = functools.partial(
        pl.kernel,
        out_type=jax.ShapeDtypeStruct((N_ROWS, N_COLS), jnp.float32),
        mesh=mesh,
        compiler_params=pltpu.CompilerParams(needs_layout_passes=False),
        scratch_types=[
            pltpu.VMEM((CHUNK_ROWS, N_COLS), jnp.float32),
            pltpu.VMEM((CHUNK_ROWS, N_COLS), jnp.float32),
            pltpu.VMEM((CHUNK_ROWS, N_COLS), jnp.float32),
            pltpu.VMEM((CHUNK_ROWS, N_COLS), jnp.float32),
            pltpu.VMEM((LANES,), jnp.float32),
            pltpu.VMEM((LANES,), jnp.float32),
            pltpu.SemaphoreType.DMA,
            pltpu.SemaphoreType.DMA,
            pltpu.SemaphoreType.DMA,
            pltpu.SemaphoreType.DMA,
        ],
    )(_sc_body)
    out = run(S.reshape(N_ROWS, N_COLS), u_i)
    return out.reshape(S.shape)
